# Initial kernel scaffold; baseline (speedup 1.0000x reference)
import jax
import jax.numpy as jnp
from jax.experimental import pallas as pl


def kernel(source_nodes, destination_nodes, edge_times, edge_idxs, n_neighbors, nbr_nodes, nbr_edges, nbr_times, params):
    raise NotImplementedError("write your pallas kernel here")


# speedup vs baseline: 2.4072x; 2.4072x over previous
import functools
import math

import jax
import jax.numpy as jnp
from jax import lax
from jax.experimental import pallas as pl
from jax.experimental.pallas import tpu as pltpu
from jax.experimental.pallas import tpu_sc as plsc

D = 128
D_E = 16
K = 10
H = 2
DQ = 2 * D
DK = 2 * D + D_E
NW = 32
CHUNK = 128
NB = 256


@functools.lru_cache(maxsize=None)
def _gather_call(v_rows, d_cols, b_pad, dtype_name):
    dtype = jnp.dtype(dtype_name)
    b_per_w = b_pad // NW
    n_chunks = b_per_w // CHUNK
    mesh = plsc.VectorSubcoreMesh(core_axis_name="c", subcore_axis_name="s")

    @functools.partial(
        pl.kernel,
        mesh=mesh,
        compiler_params=pltpu.CompilerParams(use_tc_tiling_on_sc=False),
        out_type=jax.ShapeDtypeStruct((b_pad, d_cols), dtype),
        scratch_types=[
            pltpu.VMEM((n_chunks, CHUNK), jnp.int32),
            pltpu.VMEM((2, CHUNK, d_cols), dtype),
            pltpu.SemaphoreType.DMA,
        ],
    )
    def gk(table_hbm, idx_hbm, out_hbm, idx_v, rows_v, gsem):
        wid = lax.axis_index("s") * 2 + lax.axis_index("c")
        base = wid * b_per_w
        pltpu.sync_copy(idx_hbm.at[wid], idx_v)
        pltpu.async_copy(table_hbm.at[idx_v.at[0]], rows_v.at[0], gsem)

        def body(c, carry):
            slot = lax.rem(c, 2)
            nxt = lax.rem(c + 1, 2)

            @pl.when(c + 1 < n_chunks)
            def _():
                pltpu.async_copy(table_hbm.at[idx_v.at[c + 1]], rows_v.at[nxt], gsem)

            pltpu.make_async_copy(table_hbm.at[idx_v.at[c]], rows_v.at[slot], gsem).wait()
            pltpu.sync_copy(rows_v.at[slot], out_hbm.at[pl.ds(base + c * CHUNK, CHUNK)])
            return carry

        lax.fori_loop(0, n_chunks, body, 0)

    return gk


def _sc_gather(table, idx):
    b = idx.shape[0]
    b_pad = -(-b // (NW * CHUNK)) * (NW * CHUNK)
    idx = idx.astype(jnp.int32)
    if b_pad != b:
        idx = jnp.concatenate([idx, jnp.zeros((b_pad - b,), jnp.int32)])
    idx3 = idx.reshape(NW, b_pad // NW // CHUNK, CHUNK)
    fn = _gather_call(table.shape[0], table.shape[1], b_pad, str(table.dtype))
    return fn(table, idx3)


def _attn_body(*refs, head):
    (src_ref, *rest) = refs
    nbr_refs = rest[:K]
    ef_refs = rest[K:2 * K]
    (ts_ref, nts_ref, wq_ref, wk_ref, wv_ref, wo_ref, f1w_ref, f1b_ref,
     f2w_ref, f2b_ref, tw_ref, tb_ref) = rest[2 * K:2 * K + 12]
    if head:
        hw_ref, hb_ref = rest[2 * K + 12:2 * K + 14]
        out_ref = rest[-1]
    else:
        out_ref = rest[-1]

    src = src_ref[...]
    wq = wq_ref[...]
    wk = wk_ref[...]
    wv = wv_ref[...]
    tw = tw_ref[...]
    tb = tb_ref[...]
    src_t = jnp.cos(tb)
    q = jnp.dot(src, wq[:D], preferred_element_type=jnp.float32) \
        + jnp.dot(src_t, wq[D:], preferred_element_type=jnp.float32)
    ts = ts_ref[...]
    nts = nts_ref[...]
    inv = 1.0 / math.sqrt(D)

    s0_cols, s1_cols, vks = [], [], []
    for k in range(K):
        e_t = jnp.cos((ts - nts[:, k:k + 1]) * tw + tb)
        nb_k = nbr_refs[k][...]
        ef_k = ef_refs[k][...]
        kk = (jnp.dot(nb_k, wk[:D], preferred_element_type=jnp.float32)
              + jnp.dot(e_t, wk[D:2 * D], preferred_element_type=jnp.float32)
              + jnp.dot(ef_k, wk[2 * D:], preferred_element_type=jnp.float32))
        vk = (jnp.dot(nb_k, wv[:D], preferred_element_type=jnp.float32)
              + jnp.dot(e_t, wv[D:2 * D], preferred_element_type=jnp.float32)
              + jnp.dot(ef_k, wv[2 * D:], preferred_element_type=jnp.float32))
        vks.append(vk)
        s0_cols.append(jnp.sum(kk[:, :D] * q[:, :D], axis=1, keepdims=True))
        s1_cols.append(jnp.sum(kk[:, D:] * q[:, D:], axis=1, keepdims=True))

    s0 = jnp.concatenate(s0_cols, axis=1) * inv
    s1 = jnp.concatenate(s1_cols, axis=1) * inv
    a0 = jax.nn.softmax(s0, axis=-1)
    a1 = jax.nn.softmax(s1, axis=-1)
    o0 = a0[:, 0:1] * vks[0][:, :D]
    o1 = a1[:, 0:1] * vks[0][:, D:]
    for k in range(1, K):
        o0 = o0 + a0[:, k:k + 1] * vks[k][:, :D]
        o1 = o1 + a1[:, k:k + 1] * vks[k][:, D:]

    wo = wo_ref[...]
    att = jnp.dot(o0, wo[:D], preferred_element_type=jnp.float32) \
        + jnp.dot(o1, wo[D:], preferred_element_type=jnp.float32)
    f1w = f1w_ref[...]
    h = jnp.dot(att, f1w[:DQ], preferred_element_type=jnp.float32) \
        + jnp.dot(src, f1w[DQ:], preferred_element_type=jnp.float32) + f1b_ref[...]
    h = jax.nn.relu(h)
    out = jnp.dot(h, f2w_ref[...], preferred_element_type=jnp.float32) + f2b_ref[...]
    if head:
        out_ref[...] = jax.nn.sigmoid(
            jnp.dot(out, hw_ref[...], preferred_element_type=jnp.float32) + hb_ref[...])
    else:
        out_ref[...] = out


def _attn_layer(feat, ef, ts_col, nts, weights, n, ef_base_blk, head_w=None, head_b=None):
    nb = n // NB
    wq, wk, wv, wo, f1w, f1b, f2w, f2b, tw, tb = weights
    head = head_w is not None

    in_specs = [pl.BlockSpec((NB, D), lambda i: (i, 0))]
    in_specs += [pl.BlockSpec((NB, D), (lambda i, k=k: ((k + 1) * nb + i, 0)))
                 for k in range(K)]
    in_specs += [pl.BlockSpec((NB, D_E), (lambda i, k=k: (ef_base_blk + k * nb + i, 0)))
                 for k in range(K)]
    in_specs += [
        pl.BlockSpec((NB, 1), lambda i: (i, 0)),
        pl.BlockSpec((NB, K), lambda i: (i, 0)),
        pl.BlockSpec(wq.shape, lambda i: (0, 0)),
        pl.BlockSpec(wk.shape, lambda i: (0, 0)),
        pl.BlockSpec(wv.shape, lambda i: (0, 0)),
        pl.BlockSpec(wo.shape, lambda i: (0, 0)),
        pl.BlockSpec(f1w.shape, lambda i: (0, 0)),
        pl.BlockSpec(f1b.shape, lambda i: (0, 0)),
        pl.BlockSpec(f2w.shape, lambda i: (0, 0)),
        pl.BlockSpec(f2b.shape, lambda i: (0, 0)),
        pl.BlockSpec(tw.shape, lambda i: (0, 0)),
        pl.BlockSpec(tb.shape, lambda i: (0, 0)),
    ]
    args = [feat] + [feat] * K + [ef] * K + [ts_col, nts, wq, wk, wv, wo,
                                             f1w, f1b, f2w, f2b, tw, tb]
    if head:
        in_specs += [pl.BlockSpec(head_w.shape, lambda i: (0, 0)),
                     pl.BlockSpec(head_b.shape, lambda i: (0, 0))]
        args += [head_w, head_b]
        d_out = 1
    else:
        d_out = D

    return pl.pallas_call(
        functools.partial(_attn_body, head=head),
        grid=(nb,),
        in_specs=in_specs,
        out_specs=pl.BlockSpec((NB, d_out), lambda i: (i, 0)),
        out_shape=jax.ShapeDtypeStruct((n, d_out), jnp.float32),
    )(*args)


def kernel(source_nodes, destination_nodes, edge_times, edge_idxs, n_neighbors,
           nbr_nodes, nbr_edges, nbr_times, params):
    p = params
    b2 = 2 * source_nodes.shape[0]
    nodes_all = jnp.concatenate([source_nodes, destination_nodes]).astype(jnp.int32)
    ts_all = jnp.concatenate([edge_times, edge_times])
    n_nodes = nbr_nodes.shape[0]

    combo = jnp.concatenate([
        nbr_nodes.astype(jnp.int32),
        nbr_edges.astype(jnp.int32),
        lax.bitcast_convert_type(nbr_times, jnp.int32),
        jnp.zeros((n_nodes, 32 - 3 * K), jnp.int32),
    ], axis=1)

    g2 = _sc_gather(combo, nodes_all)[:b2]
    nbrs2 = g2[:, :K]
    neids2 = g2[:, K:2 * K]
    ntimes2 = lax.bitcast_convert_type(g2[:, 2 * K:3 * K], jnp.float32)

    n1 = b2 * (1 + K)
    l1_nodes = jnp.concatenate([nodes_all, nbrs2.T.reshape(-1)])
    l1_ts = jnp.concatenate([ts_all, ntimes2.T.reshape(-1)])

    g1 = _sc_gather(combo, l1_nodes)[:n1]
    nbrs1 = g1[:, :K]
    neids1 = g1[:, K:2 * K]
    ntimes1 = lax.bitcast_convert_type(g1[:, 2 * K:3 * K], jnp.float32)

    nf_idx = jnp.concatenate([l1_nodes, nbrs1.T.reshape(-1)])
    nf = _sc_gather(p['node_features'], nf_idx)
    ef_idx = jnp.concatenate([neids1.T.reshape(-1), neids2.T.reshape(-1)])
    ef = _sc_gather(p['edge_features'], ef_idx)

    def wts(l):
        return (p['Wq%d' % l], p['Wk%d' % l], p['Wv%d' % l], p['Wo%d' % l],
                p['fc1w%d' % l], p['fc1b%d' % l].reshape(1, D),
                p['fc2w%d' % l], p['fc2b%d' % l].reshape(1, D),
                p['time_w'].reshape(1, D), p['time_b'].reshape(1, D))

    emb1 = _attn_layer(nf, ef, l1_ts.reshape(-1, 1), ntimes1, wts(0),
                       n=n1, ef_base_blk=0)
    probs = _attn_layer(emb1, ef, ts_all.reshape(-1, 1), ntimes2, wts(1),
                        n=b2, ef_base_blk=(K * n1) // NB,
                        head_w=p['w_out'], head_b=p['b_out'].reshape(1, 1))
    nsrc = source_nodes.shape[0]
    return probs[:nsrc], probs[nsrc:]
